# baseline (speedup 1.0000x reference)
import jax
import jax.numpy as jnp
from jax.experimental import pallas as pl

K = 64
ROWS_PER_BLOCK = 8
N = 32768


def _topk_block(x_ref, v_ref, i_ref):
    x = x_ref[...]
    col = jax.lax.broadcasted_iota(jnp.int32, x.shape, 1)
    out_col = jax.lax.broadcasted_iota(jnp.int32, (ROWS_PER_BLOCK, K), 1)
    big = jnp.int32(2**30)

    def step(t, carry):
        x, vals, idxs = carry
        m = jnp.max(x, axis=1, keepdims=True)
        im = jnp.min(jnp.where(x == m, col, big), axis=1, keepdims=True)
        vals = jnp.where(out_col == t, m, vals)
        idxs = jnp.where(out_col == t, im, idxs)
        x = jnp.where(col == im, -jnp.inf, x)
        return x, vals, idxs

    vals0 = jnp.zeros((ROWS_PER_BLOCK, K), jnp.float32)
    idxs0 = jnp.zeros((ROWS_PER_BLOCK, K), jnp.int32)
    _, vals, idxs = jax.lax.fori_loop(0, K, step, (x, vals0, idxs0))
    v_ref[...] = vals
    i_ref[...] = idxs


def kernel(x):
    rows = x.shape[0]
    grid = rows // ROWS_PER_BLOCK
    v, i = pl.pallas_call(
        _topk_block,
        grid=(grid,),
        in_specs=[pl.BlockSpec((ROWS_PER_BLOCK, N), lambda r: (r, 0))],
        out_specs=(
            pl.BlockSpec((ROWS_PER_BLOCK, K), lambda r: (r, 0)),
            pl.BlockSpec((ROWS_PER_BLOCK, K), lambda r: (r, 0)),
        ),
        out_shape=(
            jax.ShapeDtypeStruct((rows, K), jnp.float32),
            jax.ShapeDtypeStruct((rows, K), jnp.int32),
        ),
    )(x)
    return (v, i)


# speedup vs baseline: 6.7797x; 6.7797x over previous
import jax
import jax.numpy as jnp
from jax import lax
from jax.experimental import pallas as pl
from jax.experimental.pallas import tpu as pltpu
from jax.experimental.pallas import tpu_sc as plsc

KTOP = 64
N = 32768
ROWS = 128
LANES = 16
NV = N // LANES
GB = 16
NG = NV // GB
CL = 64
NC, NS = 2, 16
NW = NC * NS
RPW = ROWS // NW

_NEG_INF = float("-inf")
_BIG_I = 2**30


def _sc_body(x_hbm, v_hbm, i_hbm, xb, gm, vbuf, ibuf, outv, outi):
    wid = lax.axis_index("s") * NC + lax.axis_index("c")
    lane = lax.iota(jnp.int32, LANES)
    zero16i = jnp.zeros((LANES,), jnp.int32)
    neginf16 = jnp.full((LANES,), _NEG_INF, jnp.float32)
    bigi16 = jnp.full((LANES,), _BIG_I, jnp.int32)

    def do_row(r, _):
        row = wid * RPW + r
        pltpu.sync_copy(x_hbm.at[row], xb)

        def g_body(g, c):
            mn, mx = c
            m = xb[pl.ds(g * (GB * LANES), LANES)]
            for j in range(1, GB):
                m = jnp.maximum(m, xb[pl.ds(g * (GB * LANES) + j * LANES, LANES)])
            gm[pl.ds(g * LANES, LANES)] = m
            return jnp.minimum(mn, m), jnp.maximum(mx, m)

        mnv, mxv = lax.fori_loop(
            0, NG, g_body,
            (jnp.full((LANES,), jnp.inf, jnp.float32),
             jnp.full((LANES,), _NEG_INF, jnp.float32)))

        def count_gm(tvec):
            def c_body(g, acc):
                v = gm[pl.ds(g * LANES, LANES)]
                return acc + (v >= tvec).astype(jnp.int32)
            acc = lax.fori_loop(0, NG, c_body, zero16i)
            return jnp.sum(acc)

        lo0 = jnp.full((LANES,), 1.0, jnp.float32) * jnp.min(mnv)
        hi0 = jnp.full((LANES,), 1.0, jnp.float32) * jnp.max(mxv)

        def bs_cond(st):
            lo, hi, c_lo, it = st
            return (c_lo > 2 * KTOP) & (it < 30)

        def bs_body(st):
            lo, hi, c_lo, it = st
            mid = (lo + hi) * 0.5
            c = count_gm(mid)
            take = c >= KTOP
            lo = jnp.where(take, mid, lo)
            c_lo = jnp.where(take, c, c_lo)
            hi = jnp.where(take, hi, mid)
            return lo, hi, c_lo, it + jnp.int32(1)

        tvec, _, _, _ = lax.while_loop(
            bs_cond, bs_body, (lo0, hi0, jnp.int32(NG * LANES), jnp.int32(0)))

        def init_body(j, _):
            vbuf[j] = neginf16
            ibuf[j] = bigi16
            return 0
        lax.fori_loop(0, CL, init_body, 0)

        def col_body(i, cur):
            v = xb[pl.ds(i * LANES, LANES)]
            msk = (v >= tvec) & (cur < CL)
            idx = lane + i * LANES
            plsc.store_scatter(vbuf, [cur, lane], v, mask=msk)
            plsc.store_scatter(ibuf, [cur, lane], idx, mask=msk)
            return cur + msk.astype(jnp.int32)

        cursor = lax.fori_loop(0, NV, col_body, zero16i)
        lmax = jnp.max(cursor)

        def ext_body(t, st):
            ov0, ov1, ov2, ov3, oi0, oi1, oi2, oi3 = st

            def fold(j, c):
                mv, mi, mj = c
                v = vbuf[j]
                iv = ibuf[j]
                better = (v > mv) | ((v == mv) & (iv < mi))
                return (jnp.where(better, v, mv),
                        jnp.where(better, iv, mi),
                        jnp.where(better, j, mj))

            mv, mi, mj = lax.fori_loop(0, lmax, fold, (neginf16, bigi16, zero16i))
            rbest = jnp.max(mv)
            ibest = jnp.min(jnp.where(mv == rbest, mi, bigi16))
            lwin = ibest & (LANES - 1)
            jwin = jnp.max(jnp.where(lane == lwin, mj, zero16i))
            vrow = vbuf[jwin]
            vbuf[jwin] = jnp.where(lane == lwin, _NEG_INF, vrow)

            tt = t & (LANES - 1)
            tb = t >> 4
            sel = lane == tt
            ov0 = jnp.where(sel & (tb == 0), rbest, ov0)
            ov1 = jnp.where(sel & (tb == 1), rbest, ov1)
            ov2 = jnp.where(sel & (tb == 2), rbest, ov2)
            ov3 = jnp.where(sel & (tb == 3), rbest, ov3)
            oi0 = jnp.where(sel & (tb == 0), ibest, oi0)
            oi1 = jnp.where(sel & (tb == 1), ibest, oi1)
            oi2 = jnp.where(sel & (tb == 2), ibest, oi2)
            oi3 = jnp.where(sel & (tb == 3), ibest, oi3)
            return ov0, ov1, ov2, ov3, oi0, oi1, oi2, oi3

        st0 = (neginf16, neginf16, neginf16, neginf16,
               zero16i, zero16i, zero16i, zero16i)
        ov0, ov1, ov2, ov3, oi0, oi1, oi2, oi3 = lax.fori_loop(
            0, KTOP, ext_body, st0)

        outv[pl.ds(0, LANES)] = ov0
        outv[pl.ds(LANES, LANES)] = ov1
        outv[pl.ds(2 * LANES, LANES)] = ov2
        outv[pl.ds(3 * LANES, LANES)] = ov3
        outi[pl.ds(0, LANES)] = oi0
        outi[pl.ds(LANES, LANES)] = oi1
        outi[pl.ds(2 * LANES, LANES)] = oi2
        outi[pl.ds(3 * LANES, LANES)] = oi3
        pltpu.sync_copy(outv, v_hbm.at[row])
        pltpu.sync_copy(outi, i_hbm.at[row])
        return 0

    lax.fori_loop(0, RPW, do_row, 0)


def kernel(x):
    mesh = plsc.VectorSubcoreMesh(
        core_axis_name="c", subcore_axis_name="s", num_cores=NC, num_subcores=NS)
    f = pl.kernel(
        _sc_body,
        out_type=(
            jax.ShapeDtypeStruct((ROWS, KTOP), jnp.float32),
            jax.ShapeDtypeStruct((ROWS, KTOP), jnp.int32),
        ),
        mesh=mesh,
        compiler_params=pltpu.CompilerParams(needs_layout_passes=False),
        scratch_types=[
            pltpu.VMEM((N,), jnp.float32),
            pltpu.VMEM((NV,), jnp.float32),
            pltpu.VMEM((CL, LANES), jnp.float32),
            pltpu.VMEM((CL, LANES), jnp.int32),
            pltpu.VMEM((KTOP,), jnp.float32),
            pltpu.VMEM((KTOP,), jnp.int32),
        ],
    )
    return f(x)


# speedup vs baseline: 7.4931x; 1.1052x over previous
import jax
import jax.numpy as jnp
from jax import lax
from jax.experimental import pallas as pl
from jax.experimental.pallas import tpu as pltpu
from jax.experimental.pallas import tpu_sc as plsc

KTOP = 64
N = 32768
ROWS = 128
LANES = 16
NV = N // LANES
GB = 16
NG = NV // GB
CL = 64
NC, NS = 2, 16
NW = NC * NS
RPW = ROWS // NW

_NEG_INF = float("-inf")
_BIG_I = 2**30


def _sc_body(x_hbm, v_hbm, i_hbm, xb, gm, vbuf, ibuf, outv, outi):
    wid = lax.axis_index("s") * NC + lax.axis_index("c")
    lane = lax.iota(jnp.int32, LANES)
    zero16i = jnp.zeros((LANES,), jnp.int32)
    neginf16 = jnp.full((LANES,), _NEG_INF, jnp.float32)
    bigi16 = jnp.full((LANES,), _BIG_I, jnp.int32)

    def do_row(r, _):
        row = wid * RPW + r
        pltpu.sync_copy(x_hbm.at[row], xb)

        def g_body(g, c):
            mn, mx = c
            m = xb[pl.ds(g * (GB * LANES), LANES)]
            for j in range(1, GB):
                m = jnp.maximum(m, xb[pl.ds(g * (GB * LANES) + j * LANES, LANES)])
            gm[pl.ds(g * LANES, LANES)] = m
            return jnp.minimum(mn, m), jnp.maximum(mx, m)

        mnv, mxv = lax.fori_loop(
            0, NG, g_body,
            (jnp.full((LANES,), jnp.inf, jnp.float32),
             jnp.full((LANES,), _NEG_INF, jnp.float32)))

        def count_gm(tvec):
            def c_body(g, acc):
                v = gm[pl.ds(g * LANES, LANES)]
                return acc + (v >= tvec).astype(jnp.int32)
            acc = lax.fori_loop(0, NG, c_body, zero16i)
            return jnp.sum(acc)

        lo0 = jnp.full((LANES,), 1.0, jnp.float32) * jnp.min(mnv)
        hi0 = jnp.full((LANES,), 1.0, jnp.float32) * jnp.max(mxv)

        def bs_cond(st):
            lo, hi, c_lo, it = st
            return (c_lo > 2 * KTOP) & (it < 30)

        def bs_body(st):
            lo, hi, c_lo, it = st
            mid = (lo + hi) * 0.5
            c = count_gm(mid)
            take = c >= KTOP
            lo = jnp.where(take, mid, lo)
            c_lo = jnp.where(take, c, c_lo)
            hi = jnp.where(take, hi, mid)
            return lo, hi, c_lo, it + jnp.int32(1)

        tvec, _, _, _ = lax.while_loop(
            bs_cond, bs_body, (lo0, hi0, jnp.int32(NG * LANES), jnp.int32(0)))

        def init_body(j, _):
            vbuf[j] = neginf16
            return 0
        lax.fori_loop(0, CL, init_body, 0)

        UNROLL = 8

        def col_body(i0, cur):
            for u in range(UNROLL):
                i = i0 * UNROLL + u
                v = xb[pl.ds(i * LANES, LANES)]
                msk = (v >= tvec) & (cur < CL)
                idx = lane + i * LANES
                plsc.store_scatter(vbuf, [cur, lane], v, mask=msk)
                plsc.store_scatter(ibuf, [cur, lane], idx, mask=msk)
                cur = cur + msk.astype(jnp.int32)
            return cur

        cursor = lax.fori_loop(0, NV // UNROLL, col_body, zero16i)
        lmax = jnp.max(cursor)

        def sweep(_, carry):
            def ce(j, c):
                va = vbuf[j]
                vb = vbuf[j + 1]
                ia = ibuf[j]
                ib = ibuf[j + 1]
                sw = vb > va
                vbuf[j] = jnp.where(sw, vb, va)
                vbuf[j + 1] = jnp.where(sw, va, vb)
                ibuf[j] = jnp.where(sw, ib, ia)
                ibuf[j + 1] = jnp.where(sw, ia, ib)
                return c
            return lax.fori_loop(0, lmax - 1, ce, carry)

        lax.fori_loop(0, lmax, sweep, 0)

        def ext_body(t, st):
            ov0, ov1, ov2, ov3, oi0, oi1, oi2, oi3, ptr = st
            inb = ptr < CL
            hv = plsc.load_gather(vbuf, [ptr, lane], mask=inb)
            hi = plsc.load_gather(ibuf, [ptr, lane], mask=inb)
            hv = jnp.where(inb, hv, _NEG_INF)
            rbest = jnp.max(hv)
            ibest = jnp.min(jnp.where(hv == rbest, hi, bigi16))
            lwin = ibest & (LANES - 1)
            ptr = ptr + (lane == lwin).astype(jnp.int32)

            tt = t & (LANES - 1)
            tb = t >> 4
            sel = lane == tt
            ov0 = jnp.where(sel & (tb == 0), rbest, ov0)
            ov1 = jnp.where(sel & (tb == 1), rbest, ov1)
            ov2 = jnp.where(sel & (tb == 2), rbest, ov2)
            ov3 = jnp.where(sel & (tb == 3), rbest, ov3)
            oi0 = jnp.where(sel & (tb == 0), ibest, oi0)
            oi1 = jnp.where(sel & (tb == 1), ibest, oi1)
            oi2 = jnp.where(sel & (tb == 2), ibest, oi2)
            oi3 = jnp.where(sel & (tb == 3), ibest, oi3)
            return ov0, ov1, ov2, ov3, oi0, oi1, oi2, oi3, ptr

        st0 = (neginf16, neginf16, neginf16, neginf16,
               zero16i, zero16i, zero16i, zero16i, zero16i)
        ov0, ov1, ov2, ov3, oi0, oi1, oi2, oi3, _ = lax.fori_loop(
            0, KTOP, ext_body, st0)

        outv[pl.ds(0, LANES)] = ov0
        outv[pl.ds(LANES, LANES)] = ov1
        outv[pl.ds(2 * LANES, LANES)] = ov2
        outv[pl.ds(3 * LANES, LANES)] = ov3
        outi[pl.ds(0, LANES)] = oi0
        outi[pl.ds(LANES, LANES)] = oi1
        outi[pl.ds(2 * LANES, LANES)] = oi2
        outi[pl.ds(3 * LANES, LANES)] = oi3
        pltpu.sync_copy(outv, v_hbm.at[row])
        pltpu.sync_copy(outi, i_hbm.at[row])
        return 0

    lax.fori_loop(0, RPW, do_row, 0)


def kernel(x):
    mesh = plsc.VectorSubcoreMesh(
        core_axis_name="c", subcore_axis_name="s", num_cores=NC, num_subcores=NS)
    f = pl.kernel(
        _sc_body,
        out_type=(
            jax.ShapeDtypeStruct((ROWS, KTOP), jnp.float32),
            jax.ShapeDtypeStruct((ROWS, KTOP), jnp.int32),
        ),
        mesh=mesh,
        compiler_params=pltpu.CompilerParams(needs_layout_passes=False),
        scratch_types=[
            pltpu.VMEM((N,), jnp.float32),
            pltpu.VMEM((NV,), jnp.float32),
            pltpu.VMEM((CL, LANES), jnp.float32),
            pltpu.VMEM((CL, LANES), jnp.int32),
            pltpu.VMEM((KTOP,), jnp.float32),
            pltpu.VMEM((KTOP,), jnp.int32),
        ],
    )
    return f(x)


# speedup vs baseline: 8.5425x; 1.1401x over previous
import jax
import jax.numpy as jnp
from jax import lax
from jax.experimental import pallas as pl
from jax.experimental.pallas import tpu as pltpu
from jax.experimental.pallas import tpu_sc as plsc

KTOP = 64
N = 32768
ROWS = 128
LANES = 16
NV = N // LANES
GB = 16
NG = NV // GB
CL = 64
NC, NS = 2, 16
NW = NC * NS
RPW = ROWS // NW
UNROLL = 8

_NEG_INF = float("-inf")
_BIG_I = 2**30


def _sc_body(x_hbm, v_hbm, i_hbm, xb, vbuf, ibuf, outv, outi, sem):
    wid = lax.axis_index("s") * NC + lax.axis_index("c")
    lane = lax.iota(jnp.int32, LANES)
    zero16i = jnp.zeros((LANES,), jnp.int32)
    neginf16 = jnp.full((LANES,), _NEG_INF, jnp.float32)
    bigi16 = jnp.full((LANES,), _BIG_I, jnp.int32)

    row0 = wid * RPW
    pltpu.async_copy(x_hbm.at[row0], xb.at[pl.ds(0, N)], sem)

    def do_row(r, _):
        row = row0 + r
        base = (r & 1) * N
        pltpu.make_async_copy(
            x_hbm.at[row], xb.at[pl.ds(base, N)], sem).wait()

        @pl.when(r + 1 < RPW)
        def _():
            nbase = ((r + 1) & 1) * N
            pltpu.async_copy(
                x_hbm.at[row + 1], xb.at[pl.ds(nbase, N)], sem)

        def g_body(g, tops):
            t0, t1, t2, t3 = tops
            m = xb[pl.ds(base + g * (GB * LANES), LANES)]
            for j in range(1, GB):
                m = jnp.maximum(
                    m, xb[pl.ds(base + g * (GB * LANES) + j * LANES, LANES)])
            hi = jnp.maximum(t0, m)
            m = jnp.minimum(t0, m)
            t0 = hi
            hi = jnp.maximum(t1, m)
            m = jnp.minimum(t1, m)
            t1 = hi
            hi = jnp.maximum(t2, m)
            m = jnp.minimum(t2, m)
            t2 = hi
            t3 = jnp.maximum(t3, m)
            return t0, t1, t2, t3

        _, _, _, t3 = lax.fori_loop(
            0, NG, g_body, (neginf16, neginf16, neginf16, neginf16))
        tvec = jnp.zeros((LANES,), jnp.float32) + jnp.min(t3)

        def init_body(j, _):
            vbuf[j] = neginf16
            return 0
        lax.fori_loop(0, CL, init_body, 0)

        def col_body(i0, cur):
            for u in range(UNROLL):
                i = i0 * UNROLL + u
                v = xb[pl.ds(base + i * LANES, LANES)]
                msk = (v >= tvec) & (cur < CL)
                idx = lane + i * LANES
                plsc.store_scatter(ibuf, [cur, lane], idx, mask=msk)
                cur = cur + msk.astype(jnp.int32)
            return cur

        cursor = lax.fori_loop(0, NV // UNROLL, col_body, zero16i)
        lmax = jnp.max(cursor)

        def mat_body(j, _):
            idx = ibuf[j]
            ok = cursor > j
            v = plsc.load_gather(xb, [base + idx], mask=ok)
            vbuf[j] = jnp.where(ok, v, _NEG_INF)
            return 0
        lax.fori_loop(0, lmax, mat_body, 0)

        def sweep(_, carry):
            def ce(j, c):
                va = vbuf[j]
                vb = vbuf[j + 1]
                ia = ibuf[j]
                ib = ibuf[j + 1]
                sw = vb > va
                vbuf[j] = jnp.where(sw, vb, va)
                vbuf[j + 1] = jnp.where(sw, va, vb)
                ibuf[j] = jnp.where(sw, ib, ia)
                ibuf[j + 1] = jnp.where(sw, ia, ib)
                return c
            return lax.fori_loop(0, lmax - 1, ce, carry)

        lax.fori_loop(0, lmax, sweep, 0)

        def ext_body(t, st):
            ov0, ov1, ov2, ov3, oi0, oi1, oi2, oi3, ptr = st
            inb = ptr < CL
            hv = plsc.load_gather(vbuf, [ptr, lane], mask=inb)
            hi = plsc.load_gather(ibuf, [ptr, lane], mask=inb)
            hv = jnp.where(inb, hv, _NEG_INF)
            rbest = jnp.max(hv)
            ibest = jnp.min(jnp.where(hv == rbest, hi, bigi16))
            lwin = ibest & (LANES - 1)
            ptr = ptr + (lane == lwin).astype(jnp.int32)

            tt = t & (LANES - 1)
            tb = t >> 4
            sel = lane == tt
            ov0 = jnp.where(sel & (tb == 0), rbest, ov0)
            ov1 = jnp.where(sel & (tb == 1), rbest, ov1)
            ov2 = jnp.where(sel & (tb == 2), rbest, ov2)
            ov3 = jnp.where(sel & (tb == 3), rbest, ov3)
            oi0 = jnp.where(sel & (tb == 0), ibest, oi0)
            oi1 = jnp.where(sel & (tb == 1), ibest, oi1)
            oi2 = jnp.where(sel & (tb == 2), ibest, oi2)
            oi3 = jnp.where(sel & (tb == 3), ibest, oi3)
            return ov0, ov1, ov2, ov3, oi0, oi1, oi2, oi3, ptr

        st0 = (neginf16, neginf16, neginf16, neginf16,
               zero16i, zero16i, zero16i, zero16i, zero16i)
        ov0, ov1, ov2, ov3, oi0, oi1, oi2, oi3, _ = lax.fori_loop(
            0, KTOP, ext_body, st0)

        outv[pl.ds(0, LANES)] = ov0
        outv[pl.ds(LANES, LANES)] = ov1
        outv[pl.ds(2 * LANES, LANES)] = ov2
        outv[pl.ds(3 * LANES, LANES)] = ov3
        outi[pl.ds(0, LANES)] = oi0
        outi[pl.ds(LANES, LANES)] = oi1
        outi[pl.ds(2 * LANES, LANES)] = oi2
        outi[pl.ds(3 * LANES, LANES)] = oi3
        pltpu.sync_copy(outv, v_hbm.at[row])
        pltpu.sync_copy(outi, i_hbm.at[row])
        return 0

    lax.fori_loop(0, RPW, do_row, 0)


def kernel(x):
    mesh = plsc.VectorSubcoreMesh(
        core_axis_name="c", subcore_axis_name="s", num_cores=NC, num_subcores=NS)
    f = pl.kernel(
        _sc_body,
        out_type=(
            jax.ShapeDtypeStruct((ROWS, KTOP), jnp.float32),
            jax.ShapeDtypeStruct((ROWS, KTOP), jnp.int32),
        ),
        mesh=mesh,
        compiler_params=pltpu.CompilerParams(needs_layout_passes=False),
        scratch_types=[
            pltpu.VMEM((2 * N,), jnp.float32),
            pltpu.VMEM((CL, LANES), jnp.float32),
            pltpu.VMEM((CL, LANES), jnp.int32),
            pltpu.VMEM((KTOP,), jnp.float32),
            pltpu.VMEM((KTOP,), jnp.int32),
            pltpu.SemaphoreType.DMA,
        ],
    )
    return f(x)


# speedup vs baseline: 11.4494x; 1.3403x over previous
import jax
import jax.numpy as jnp
from jax import lax
from jax.experimental import pallas as pl
from jax.experimental.pallas import tpu as pltpu
from jax.experimental.pallas import tpu_sc as plsc

KTOP = 64
N = 32768
ROWS = 128
LANES = 16
NV = N // LANES
GB = 16
NG = NV // GB
NCH = 4
CL = 32
NC, NS = 2, 16
NW = NC * NS
RPW = ROWS // NW
UNROLL = 8

_NEG_INF = float("-inf")
_BIG_I = 2**30


def _sc_body(x_hbm, v_hbm, i_hbm, xb, vbuf, ibuf, outv, outi, sem):
    wid = lax.axis_index("s") * NC + lax.axis_index("c")
    lane = lax.iota(jnp.int32, LANES)
    zero16i = jnp.zeros((LANES,), jnp.int32)
    neginf16 = jnp.full((LANES,), _NEG_INF, jnp.float32)
    bigi16 = jnp.full((LANES,), _BIG_I, jnp.int32)

    row0 = wid * RPW
    pltpu.async_copy(x_hbm.at[pl.ds(row0 * N, N)], xb.at[pl.ds(0, N)], sem)

    def do_row(r, _):
        row = row0 + r
        base = (r & 1) * N
        pltpu.make_async_copy(
            x_hbm.at[pl.ds(row * N, N)], xb.at[pl.ds(base, N)], sem).wait()

        @pl.when(r + 1 < RPW)
        def _():
            nbase = ((r + 1) & 1) * N
            pltpu.async_copy(
                x_hbm.at[pl.ds((row + 1) * N, N)], xb.at[pl.ds(nbase, N)], sem)

        def g_body(g, tops):
            t0, t1, t2, t3 = tops
            m = xb[pl.ds(base + g * (GB * LANES), LANES)]
            for j in range(1, GB):
                m = jnp.maximum(
                    m, xb[pl.ds(base + g * (GB * LANES) + j * LANES, LANES)])
            hi = jnp.maximum(t0, m)
            m = jnp.minimum(t0, m)
            t0 = hi
            hi = jnp.maximum(t1, m)
            m = jnp.minimum(t1, m)
            t1 = hi
            hi = jnp.maximum(t2, m)
            m = jnp.minimum(t2, m)
            t2 = hi
            t3 = jnp.maximum(t3, m)
            return t0, t1, t2, t3

        _, _, _, t3 = lax.fori_loop(
            0, NG, g_body, (neginf16, neginf16, neginf16, neginf16))
        tvec = jnp.zeros((LANES,), jnp.float32) + jnp.min(t3)

        def init_body(j, _):
            vbuf[j] = neginf16
            return 0
        lax.fori_loop(0, NCH * CL, init_body, 0)

        def col_body(i0, curs):
            curs = list(curs)
            bi = i0 * UNROLL
            vs = [xb[pl.ds(base + (bi + u) * LANES, LANES)]
                  for u in range(UNROLL)]
            for u in range(UNROLL):
                ch = u & (NCH - 1)
                v = vs[u]
                msk = (v >= tvec) & (curs[ch] < CL)
                idx = lane + (bi + u) * LANES
                plsc.store_scatter(
                    ibuf, [curs[ch] + ch * CL, lane], idx, mask=msk)
                curs[ch] = curs[ch] + msk.astype(jnp.int32)
            return tuple(curs)

        curs = lax.fori_loop(0, NV // UNROLL, col_body, (zero16i,) * NCH)

        for ch in range(NCH):
            cur_c = curs[ch]
            lmax_c = jnp.max(cur_c)

            def mat_body(j, _, ch=ch, cur_c=cur_c):
                idx = ibuf[ch * CL + j]
                ok = cur_c > j
                v = plsc.load_gather(xb, [base + idx], mask=ok)
                vbuf[ch * CL + j] = jnp.where(ok, v, _NEG_INF)
                return 0

            lax.fori_loop(0, lmax_c, mat_body, 0)

            def sweep(_, carry, ch=ch, lmax_c=lmax_c):
                def ce(j, c):
                    a = ch * CL + j
                    va = vbuf[a]
                    vb = vbuf[a + 1]
                    ia = ibuf[a]
                    ib = ibuf[a + 1]
                    sw = vb > va
                    vbuf[a] = jnp.where(sw, vb, va)
                    vbuf[a + 1] = jnp.where(sw, va, vb)
                    ibuf[a] = jnp.where(sw, ib, ia)
                    ibuf[a + 1] = jnp.where(sw, ia, ib)
                    return c
                return lax.fori_loop(0, lmax_c - 1, ce, carry)

            lax.fori_loop(0, lmax_c, sweep, 0)

        def ext_body(t, st):
            (ov0, ov1, ov2, ov3, oi0, oi1, oi2, oi3,
             p0, p1, p2, p3) = st
            ptrs = [p0, p1, p2, p3]
            hvs, his = [], []
            for ch in range(NCH):
                inb = ptrs[ch] < CL
                hv = plsc.load_gather(vbuf, [ptrs[ch] + ch * CL, lane],
                                      mask=inb)
                hi = plsc.load_gather(ibuf, [ptrs[ch] + ch * CL, lane],
                                      mask=inb)
                hvs.append(jnp.where(inb, hv, _NEG_INF))
                his.append(hi)
            hvm = jnp.maximum(jnp.maximum(hvs[0], hvs[1]),
                              jnp.maximum(hvs[2], hvs[3]))
            rbest = jnp.max(hvm)
            mm = jnp.minimum(
                jnp.minimum(jnp.where(hvs[0] == rbest, his[0], bigi16),
                            jnp.where(hvs[1] == rbest, his[1], bigi16)),
                jnp.minimum(jnp.where(hvs[2] == rbest, his[2], bigi16),
                            jnp.where(hvs[3] == rbest, his[3], bigi16)))
            ibest = jnp.min(mm)
            lwin = ibest & (LANES - 1)
            selw = lane == lwin
            for ch in range(NCH):
                upd = selw & (hvs[ch] == rbest) & (his[ch] == ibest)
                ptrs[ch] = ptrs[ch] + upd.astype(jnp.int32)

            tt = t & (LANES - 1)
            tb = t >> 4
            sel = lane == tt
            ov0 = jnp.where(sel & (tb == 0), rbest, ov0)
            ov1 = jnp.where(sel & (tb == 1), rbest, ov1)
            ov2 = jnp.where(sel & (tb == 2), rbest, ov2)
            ov3 = jnp.where(sel & (tb == 3), rbest, ov3)
            oi0 = jnp.where(sel & (tb == 0), ibest, oi0)
            oi1 = jnp.where(sel & (tb == 1), ibest, oi1)
            oi2 = jnp.where(sel & (tb == 2), ibest, oi2)
            oi3 = jnp.where(sel & (tb == 3), ibest, oi3)
            return (ov0, ov1, ov2, ov3, oi0, oi1, oi2, oi3,
                    ptrs[0], ptrs[1], ptrs[2], ptrs[3])

        st0 = (neginf16, neginf16, neginf16, neginf16,
               zero16i, zero16i, zero16i, zero16i,
               zero16i, zero16i, zero16i, zero16i)
        out = lax.fori_loop(0, KTOP, ext_body, st0)
        ov0, ov1, ov2, ov3, oi0, oi1, oi2, oi3 = out[:8]

        outv[pl.ds(0, LANES)] = ov0
        outv[pl.ds(LANES, LANES)] = ov1
        outv[pl.ds(2 * LANES, LANES)] = ov2
        outv[pl.ds(3 * LANES, LANES)] = ov3
        outi[pl.ds(0, LANES)] = oi0
        outi[pl.ds(LANES, LANES)] = oi1
        outi[pl.ds(2 * LANES, LANES)] = oi2
        outi[pl.ds(3 * LANES, LANES)] = oi3
        pltpu.sync_copy(outv, v_hbm.at[row])
        pltpu.sync_copy(outi, i_hbm.at[row])
        return 0

    lax.fori_loop(0, RPW, do_row, 0)


def kernel(x):
    mesh = plsc.VectorSubcoreMesh(
        core_axis_name="c", subcore_axis_name="s", num_cores=NC, num_subcores=NS)
    f = pl.kernel(
        _sc_body,
        out_type=(
            jax.ShapeDtypeStruct((ROWS, KTOP), jnp.float32),
            jax.ShapeDtypeStruct((ROWS, KTOP), jnp.int32),
        ),
        mesh=mesh,
        compiler_params=pltpu.CompilerParams(needs_layout_passes=False),
        scratch_types=[
            pltpu.VMEM((2 * N,), jnp.float32),
            pltpu.VMEM((NCH * CL, LANES), jnp.float32),
            pltpu.VMEM((NCH * CL, LANES), jnp.int32),
            pltpu.VMEM((KTOP,), jnp.float32),
            pltpu.VMEM((KTOP,), jnp.int32),
            pltpu.SemaphoreType.DMA,
        ],
    )
    return f(x.reshape(ROWS * N))


# speedup vs baseline: 14.4312x; 1.2604x over previous
import jax
import jax.numpy as jnp
from jax import lax
from jax.experimental import pallas as pl
from jax.experimental.pallas import tpu as pltpu
from jax.experimental.pallas import tpu_sc as plsc

KTOP = 64
N = 32768
ROWS = 128
LANES = 16
NV = N // LANES
GB = 16
NG = NV // GB
NCH = 4
CL = 32
NC, NS = 2, 16
NW = NC * NS
RPW = ROWS // NW
UNROLL = 8

_NEG_INF = float("-inf")
_BIG_I = 2**30


def _sc_body(x_hbm, v_hbm, i_hbm, xb, vbuf, ibuf, outv, outi, sem):
    wid = lax.axis_index("s") * NC + lax.axis_index("c")
    lane = lax.iota(jnp.int32, LANES)
    zero16i = jnp.zeros((LANES,), jnp.int32)
    neginf16 = jnp.full((LANES,), _NEG_INF, jnp.float32)
    bigi16 = jnp.full((LANES,), _BIG_I, jnp.int32)

    row0 = wid * RPW
    pltpu.async_copy(x_hbm.at[row0], xb.at[pl.ds(0, N)], sem)

    def do_row(r, _):
        row = row0 + r
        base = (r & 1) * N
        pltpu.make_async_copy(
            x_hbm.at[row], xb.at[pl.ds(base, N)], sem).wait()

        @pl.when(r + 1 < RPW)
        def _():
            nbase = ((r + 1) & 1) * N
            pltpu.async_copy(
                x_hbm.at[row + 1], xb.at[pl.ds(nbase, N)], sem)

        def g_body(g, tops):
            t0, t1, t2, t3 = tops
            m = xb[pl.ds(base + g * (GB * LANES), LANES)]
            for j in range(1, GB):
                m = jnp.maximum(
                    m, xb[pl.ds(base + g * (GB * LANES) + j * LANES, LANES)])
            hi = jnp.maximum(t0, m)
            m = jnp.minimum(t0, m)
            t0 = hi
            hi = jnp.maximum(t1, m)
            m = jnp.minimum(t1, m)
            t1 = hi
            hi = jnp.maximum(t2, m)
            m = jnp.minimum(t2, m)
            t2 = hi
            t3 = jnp.maximum(t3, m)
            return t0, t1, t2, t3

        _, _, _, t3 = lax.fori_loop(
            0, NG, g_body, (neginf16, neginf16, neginf16, neginf16))
        tvec = jnp.zeros((LANES,), jnp.float32) + jnp.min(t3)

        def init_body(j, _):
            vbuf[j] = neginf16
            return 0
        lax.fori_loop(0, NCH * CL, init_body, 0)

        def col_body(i0, curs):
            curs = list(curs)
            bi = i0 * UNROLL
            vs = [xb[pl.ds(base + (bi + u) * LANES, LANES)]
                  for u in range(UNROLL)]
            for u in range(UNROLL):
                ch = u & (NCH - 1)
                v = vs[u]
                msk = (v >= tvec) & (curs[ch] < CL)
                idx = lane + (bi + u) * LANES
                plsc.store_scatter(
                    ibuf, [curs[ch] + ch * CL, lane], idx, mask=msk)
                curs[ch] = curs[ch] + msk.astype(jnp.int32)
            return tuple(curs)

        curs = lax.fori_loop(0, NV // UNROLL, col_body, (zero16i,) * NCH)

        for ch in range(NCH):
            cur_c = curs[ch]
            lmax_c = jnp.max(cur_c)

            def mat_body(j, _, ch=ch, cur_c=cur_c):
                idx = ibuf[ch * CL + j]
                ok = cur_c > j
                v = plsc.load_gather(xb, [base + idx], mask=ok)
                vbuf[ch * CL + j] = jnp.where(ok, v, _NEG_INF)
                return 0

            lax.fori_loop(0, lmax_c, mat_body, 0)

            def sweep(_, carry, ch=ch, lmax_c=lmax_c):
                def ce(j, c):
                    a = ch * CL + j
                    va = vbuf[a]
                    vb = vbuf[a + 1]
                    ia = ibuf[a]
                    ib = ibuf[a + 1]
                    sw = vb > va
                    vbuf[a] = jnp.where(sw, vb, va)
                    vbuf[a + 1] = jnp.where(sw, va, vb)
                    ibuf[a] = jnp.where(sw, ib, ia)
                    ibuf[a + 1] = jnp.where(sw, ia, ib)
                    return c
                return lax.fori_loop(0, lmax_c - 1, ce, carry)

            lax.fori_loop(0, lmax_c, sweep, 0)

        def ext_body(t, st):
            (ov0, ov1, ov2, ov3, oi0, oi1, oi2, oi3,
             p0, p1, p2, p3) = st
            ptrs = [p0, p1, p2, p3]
            hvs, his = [], []
            for ch in range(NCH):
                inb = ptrs[ch] < CL
                hv = plsc.load_gather(vbuf, [ptrs[ch] + ch * CL, lane],
                                      mask=inb)
                hi = plsc.load_gather(ibuf, [ptrs[ch] + ch * CL, lane],
                                      mask=inb)
                hvs.append(jnp.where(inb, hv, _NEG_INF))
                his.append(hi)
            hvm = jnp.maximum(jnp.maximum(hvs[0], hvs[1]),
                              jnp.maximum(hvs[2], hvs[3]))
            rbest = jnp.max(hvm)
            mm = jnp.minimum(
                jnp.minimum(jnp.where(hvs[0] == rbest, his[0], bigi16),
                            jnp.where(hvs[1] == rbest, his[1], bigi16)),
                jnp.minimum(jnp.where(hvs[2] == rbest, his[2], bigi16),
                            jnp.where(hvs[3] == rbest, his[3], bigi16)))
            ibest = jnp.min(mm)
            lwin = ibest & (LANES - 1)
            selw = lane == lwin
            for ch in range(NCH):
                upd = selw & (hvs[ch] == rbest) & (his[ch] == ibest)
                ptrs[ch] = ptrs[ch] + upd.astype(jnp.int32)

            tt = t & (LANES - 1)
            tb = t >> 4
            sel = lane == tt
            ov0 = jnp.where(sel & (tb == 0), rbest, ov0)
            ov1 = jnp.where(sel & (tb == 1), rbest, ov1)
            ov2 = jnp.where(sel & (tb == 2), rbest, ov2)
            ov3 = jnp.where(sel & (tb == 3), rbest, ov3)
            oi0 = jnp.where(sel & (tb == 0), ibest, oi0)
            oi1 = jnp.where(sel & (tb == 1), ibest, oi1)
            oi2 = jnp.where(sel & (tb == 2), ibest, oi2)
            oi3 = jnp.where(sel & (tb == 3), ibest, oi3)
            return (ov0, ov1, ov2, ov3, oi0, oi1, oi2, oi3,
                    ptrs[0], ptrs[1], ptrs[2], ptrs[3])

        st0 = (neginf16, neginf16, neginf16, neginf16,
               zero16i, zero16i, zero16i, zero16i,
               zero16i, zero16i, zero16i, zero16i)
        out = lax.fori_loop(0, KTOP, ext_body, st0)
        ov0, ov1, ov2, ov3, oi0, oi1, oi2, oi3 = out[:8]

        outv[pl.ds(0, LANES)] = ov0
        outv[pl.ds(LANES, LANES)] = ov1
        outv[pl.ds(2 * LANES, LANES)] = ov2
        outv[pl.ds(3 * LANES, LANES)] = ov3
        outi[pl.ds(0, LANES)] = oi0
        outi[pl.ds(LANES, LANES)] = oi1
        outi[pl.ds(2 * LANES, LANES)] = oi2
        outi[pl.ds(3 * LANES, LANES)] = oi3
        pltpu.sync_copy(outv, v_hbm.at[row])
        pltpu.sync_copy(outi, i_hbm.at[row])
        return 0

    lax.fori_loop(0, RPW, do_row, 0)


def kernel(x):
    mesh = plsc.VectorSubcoreMesh(
        core_axis_name="c", subcore_axis_name="s", num_cores=NC, num_subcores=NS)
    f = pl.kernel(
        _sc_body,
        out_type=(
            jax.ShapeDtypeStruct((ROWS, KTOP), jnp.float32),
            jax.ShapeDtypeStruct((ROWS, KTOP), jnp.int32),
        ),
        mesh=mesh,
        compiler_params=pltpu.CompilerParams(needs_layout_passes=False, use_tc_tiling_on_sc=True),
        scratch_types=[
            pltpu.VMEM((2 * N,), jnp.float32),
            pltpu.VMEM((NCH * CL, LANES), jnp.float32),
            pltpu.VMEM((NCH * CL, LANES), jnp.int32),
            pltpu.VMEM((KTOP,), jnp.float32),
            pltpu.VMEM((KTOP,), jnp.int32),
            pltpu.SemaphoreType.DMA,
        ],
    )
    return f(x)


# speedup vs baseline: 14.4982x; 1.0046x over previous
import jax
import jax.numpy as jnp
from jax import lax
from jax.experimental import pallas as pl
from jax.experimental.pallas import tpu as pltpu
from jax.experimental.pallas import tpu_sc as plsc

KTOP = 64
N = 32768
ROWS = 128
LANES = 16
NV = N // LANES
GB = 32
NG = NV // GB
NCH = 4
CL = 32
NC, NS = 2, 16
NW = NC * NS
RPW = ROWS // NW
UNROLL = 16

_NEG_INF = float("-inf")
_BIG_I = 2**30


def _sc_body(x_hbm, v_hbm, i_hbm, xb, vbuf, ibuf, outv, outi, sem):
    wid = lax.axis_index("s") * NC + lax.axis_index("c")
    lane = lax.iota(jnp.int32, LANES)
    zero16i = jnp.zeros((LANES,), jnp.int32)
    neginf16 = jnp.full((LANES,), _NEG_INF, jnp.float32)
    bigi16 = jnp.full((LANES,), _BIG_I, jnp.int32)

    row0 = wid * RPW
    pltpu.async_copy(x_hbm.at[row0], xb.at[pl.ds(0, N)], sem)

    def do_row(r, _):
        row = row0 + r
        base = (r & 1) * N
        pltpu.make_async_copy(
            x_hbm.at[row], xb.at[pl.ds(base, N)], sem).wait()

        @pl.when(r + 1 < RPW)
        def _():
            nbase = ((r + 1) & 1) * N
            pltpu.async_copy(
                x_hbm.at[row + 1], xb.at[pl.ds(nbase, N)], sem)

        def g_body(g, tops):
            t0, t1, t2, t3 = tops
            m = xb[pl.ds(base + g * (GB * LANES), LANES)]
            for j in range(1, GB):
                m = jnp.maximum(
                    m, xb[pl.ds(base + g * (GB * LANES) + j * LANES, LANES)])
            hi = jnp.maximum(t0, m)
            m = jnp.minimum(t0, m)
            t0 = hi
            hi = jnp.maximum(t1, m)
            m = jnp.minimum(t1, m)
            t1 = hi
            hi = jnp.maximum(t2, m)
            m = jnp.minimum(t2, m)
            t2 = hi
            t3 = jnp.maximum(t3, m)
            return t0, t1, t2, t3

        _, _, _, t3 = lax.fori_loop(
            0, NG, g_body, (neginf16, neginf16, neginf16, neginf16))
        tvec = jnp.zeros((LANES,), jnp.float32) + jnp.min(t3)

        def init_body(j, _):
            vbuf[j] = neginf16
            return 0
        lax.fori_loop(0, NCH * CL, init_body, 0)

        def col_body(i0, curs):
            curs = list(curs)
            bi = i0 * UNROLL
            vs = [xb[pl.ds(base + (bi + u) * LANES, LANES)]
                  for u in range(UNROLL)]
            for u in range(UNROLL):
                ch = u & (NCH - 1)
                v = vs[u]
                msk = v >= tvec
                idx = lane + (bi + u) * LANES
                addr = jnp.minimum(curs[ch], CL - 1) + ch * CL
                plsc.store_scatter(ibuf, [addr, lane], idx, mask=msk)
                curs[ch] = curs[ch] + msk.astype(jnp.int32)
            return tuple(curs)

        curs = lax.fori_loop(0, NV // UNROLL, col_body, (zero16i,) * NCH)

        for ch in range(NCH):
            cur_c = curs[ch]
            lmax_c = jnp.minimum(jnp.max(cur_c), CL)

            def mat_body(j, _, ch=ch, cur_c=cur_c):
                idx = ibuf[ch * CL + j]
                ok = cur_c > j
                v = plsc.load_gather(xb, [base + idx], mask=ok)
                vbuf[ch * CL + j] = jnp.where(ok, v, _NEG_INF)
                return 0

            lax.fori_loop(0, lmax_c, mat_body, 0)

            def sweep(_, carry, ch=ch, lmax_c=lmax_c):
                def ce(j, c):
                    a = ch * CL + j
                    va = vbuf[a]
                    vb = vbuf[a + 1]
                    ia = ibuf[a]
                    ib = ibuf[a + 1]
                    sw = vb > va
                    vbuf[a] = jnp.where(sw, vb, va)
                    vbuf[a + 1] = jnp.where(sw, va, vb)
                    ibuf[a] = jnp.where(sw, ib, ia)
                    ibuf[a + 1] = jnp.where(sw, ia, ib)
                    return c
                return lax.fori_loop(0, lmax_c - 1, ce, carry)

            lax.fori_loop(0, lmax_c, sweep, 0)

        def ext_body(t, st):
            (ov0, ov1, ov2, ov3, oi0, oi1, oi2, oi3,
             p0, p1, p2, p3) = st
            ptrs = [p0, p1, p2, p3]
            hvs, his = [], []
            for ch in range(NCH):
                inb = ptrs[ch] < CL
                hv = plsc.load_gather(vbuf, [ptrs[ch] + ch * CL, lane],
                                      mask=inb)
                hi = plsc.load_gather(ibuf, [ptrs[ch] + ch * CL, lane],
                                      mask=inb)
                hvs.append(jnp.where(inb, hv, _NEG_INF))
                his.append(hi)
            hvm = jnp.maximum(jnp.maximum(hvs[0], hvs[1]),
                              jnp.maximum(hvs[2], hvs[3]))
            rbest = jnp.max(hvm)
            mm = jnp.minimum(
                jnp.minimum(jnp.where(hvs[0] == rbest, his[0], bigi16),
                            jnp.where(hvs[1] == rbest, his[1], bigi16)),
                jnp.minimum(jnp.where(hvs[2] == rbest, his[2], bigi16),
                            jnp.where(hvs[3] == rbest, his[3], bigi16)))
            ibest = jnp.min(mm)
            lwin = ibest & (LANES - 1)
            selw = lane == lwin
            for ch in range(NCH):
                upd = selw & (hvs[ch] == rbest) & (his[ch] == ibest)
                ptrs[ch] = ptrs[ch] + upd.astype(jnp.int32)

            tt = t & (LANES - 1)
            tb = t >> 4
            sel = lane == tt
            ov0 = jnp.where(sel & (tb == 0), rbest, ov0)
            ov1 = jnp.where(sel & (tb == 1), rbest, ov1)
            ov2 = jnp.where(sel & (tb == 2), rbest, ov2)
            ov3 = jnp.where(sel & (tb == 3), rbest, ov3)
            oi0 = jnp.where(sel & (tb == 0), ibest, oi0)
            oi1 = jnp.where(sel & (tb == 1), ibest, oi1)
            oi2 = jnp.where(sel & (tb == 2), ibest, oi2)
            oi3 = jnp.where(sel & (tb == 3), ibest, oi3)
            return (ov0, ov1, ov2, ov3, oi0, oi1, oi2, oi3,
                    ptrs[0], ptrs[1], ptrs[2], ptrs[3])

        st0 = (neginf16, neginf16, neginf16, neginf16,
               zero16i, zero16i, zero16i, zero16i,
               zero16i, zero16i, zero16i, zero16i)
        out = lax.fori_loop(0, KTOP, ext_body, st0)
        ov0, ov1, ov2, ov3, oi0, oi1, oi2, oi3 = out[:8]

        outv[pl.ds(0, LANES)] = ov0
        outv[pl.ds(LANES, LANES)] = ov1
        outv[pl.ds(2 * LANES, LANES)] = ov2
        outv[pl.ds(3 * LANES, LANES)] = ov3
        outi[pl.ds(0, LANES)] = oi0
        outi[pl.ds(LANES, LANES)] = oi1
        outi[pl.ds(2 * LANES, LANES)] = oi2
        outi[pl.ds(3 * LANES, LANES)] = oi3
        pltpu.sync_copy(outv, v_hbm.at[row])
        pltpu.sync_copy(outi, i_hbm.at[row])
        return 0

    lax.fori_loop(0, RPW, do_row, 0)


def kernel(x):
    mesh = plsc.VectorSubcoreMesh(
        core_axis_name="c", subcore_axis_name="s", num_cores=NC, num_subcores=NS)
    f = pl.kernel(
        _sc_body,
        out_type=(
            jax.ShapeDtypeStruct((ROWS, KTOP), jnp.float32),
            jax.ShapeDtypeStruct((ROWS, KTOP), jnp.int32),
        ),
        mesh=mesh,
        compiler_params=pltpu.CompilerParams(needs_layout_passes=False, use_tc_tiling_on_sc=True),
        scratch_types=[
            pltpu.VMEM((2 * N,), jnp.float32),
            pltpu.VMEM((NCH * CL, LANES), jnp.float32),
            pltpu.VMEM((NCH * CL, LANES), jnp.int32),
            pltpu.VMEM((KTOP,), jnp.float32),
            pltpu.VMEM((KTOP,), jnp.int32),
            pltpu.SemaphoreType.DMA,
        ],
    )
    return f(x)


# speedup vs baseline: 14.7431x; 1.0169x over previous
import jax
import jax.numpy as jnp
from jax import lax
from jax.experimental import pallas as pl
from jax.experimental.pallas import tpu as pltpu
from jax.experimental.pallas import tpu_sc as plsc

KTOP = 64
N = 32768
ROWS = 128
LANES = 16
NV = N // LANES
GB = 64
NG = NV // GB
NCH = 4
CL = 32
NC, NS = 2, 16
NW = NC * NS
RPW = ROWS // NW
UNROLL = 16

_NEG_INF = float("-inf")
_BIG_I = 2**30


def _sc_body(x_hbm, v_hbm, i_hbm, xb, vbuf, ibuf, outv, outi, sem):
    wid = lax.axis_index("s") * NC + lax.axis_index("c")
    lane = lax.iota(jnp.int32, LANES)
    zero16i = jnp.zeros((LANES,), jnp.int32)
    neginf16 = jnp.full((LANES,), _NEG_INF, jnp.float32)
    bigi16 = jnp.full((LANES,), _BIG_I, jnp.int32)

    row0 = wid * RPW
    pltpu.async_copy(x_hbm.at[row0], xb.at[pl.ds(0, N)], sem)

    def do_row(r, _):
        row = row0 + r
        base = (r & 1) * N
        pltpu.make_async_copy(
            x_hbm.at[row], xb.at[pl.ds(base, N)], sem).wait()

        @pl.when(r + 1 < RPW)
        def _():
            nbase = ((r + 1) & 1) * N
            pltpu.async_copy(
                x_hbm.at[row + 1], xb.at[pl.ds(nbase, N)], sem)

        def g_body(g, tops):
            t0, t1, t2, t3 = tops
            m = xb[pl.ds(base + g * (GB * LANES), LANES)]
            for j in range(1, GB):
                m = jnp.maximum(
                    m, xb[pl.ds(base + g * (GB * LANES) + j * LANES, LANES)])
            hi = jnp.maximum(t0, m)
            m = jnp.minimum(t0, m)
            t0 = hi
            hi = jnp.maximum(t1, m)
            m = jnp.minimum(t1, m)
            t1 = hi
            hi = jnp.maximum(t2, m)
            m = jnp.minimum(t2, m)
            t2 = hi
            t3 = jnp.maximum(t3, m)
            return t0, t1, t2, t3

        _, _, _, t3 = lax.fori_loop(
            0, NG, g_body, (neginf16, neginf16, neginf16, neginf16))
        tvec = jnp.zeros((LANES,), jnp.float32) + jnp.min(t3)

        def init_body(j, _):
            vbuf[j] = neginf16
            return 0
        lax.fori_loop(0, NCH * CL, init_body, 0)

        def col_body(i0, curs):
            curs = list(curs)
            bi = i0 * UNROLL
            vs = [xb[pl.ds(base + (bi + u) * LANES, LANES)]
                  for u in range(UNROLL)]
            for u in range(UNROLL):
                ch = u & (NCH - 1)
                v = vs[u]
                msk = v >= tvec
                idx = lane + (bi + u) * LANES
                addr = (curs[ch] & (CL - 1)) + ch * CL
                plsc.store_scatter(ibuf, [addr, lane], idx, mask=msk)
                curs[ch] = curs[ch] + msk.astype(jnp.int32)
            return tuple(curs)

        curs = lax.fori_loop(0, NV // UNROLL, col_body, (zero16i,) * NCH)

        for ch in range(NCH):
            cur_c = curs[ch]
            lmax_c = jnp.minimum(jnp.max(cur_c), CL)

            def mat_body(j, _, ch=ch, cur_c=cur_c):
                idx = ibuf[ch * CL + j]
                ok = cur_c > j
                v = plsc.load_gather(xb, [base + idx], mask=ok)
                vbuf[ch * CL + j] = jnp.where(ok, v, _NEG_INF)
                return 0

            lax.fori_loop(0, lmax_c, mat_body, 0)

            def sweep(_, carry, ch=ch, lmax_c=lmax_c):
                def ce(j, c):
                    a = ch * CL + j
                    va = vbuf[a]
                    vb = vbuf[a + 1]
                    ia = ibuf[a]
                    ib = ibuf[a + 1]
                    sw = vb > va
                    vbuf[a] = jnp.where(sw, vb, va)
                    vbuf[a + 1] = jnp.where(sw, va, vb)
                    ibuf[a] = jnp.where(sw, ib, ia)
                    ibuf[a + 1] = jnp.where(sw, ia, ib)
                    return c
                return lax.fori_loop(0, lmax_c - 1, ce, carry)

            lax.fori_loop(0, lmax_c, sweep, 0)

        def ext_body(t, st):
            (ov0, ov1, ov2, ov3, oi0, oi1, oi2, oi3,
             p0, p1, p2, p3) = st
            ptrs = [p0, p1, p2, p3]
            hvs, his = [], []
            for ch in range(NCH):
                inb = ptrs[ch] < CL
                hv = plsc.load_gather(vbuf, [ptrs[ch] + ch * CL, lane],
                                      mask=inb)
                hi = plsc.load_gather(ibuf, [ptrs[ch] + ch * CL, lane],
                                      mask=inb)
                hvs.append(jnp.where(inb, hv, _NEG_INF))
                his.append(hi)
            hvm = jnp.maximum(jnp.maximum(hvs[0], hvs[1]),
                              jnp.maximum(hvs[2], hvs[3]))
            rbest = jnp.max(hvm)
            mm = jnp.minimum(
                jnp.minimum(jnp.where(hvs[0] == rbest, his[0], bigi16),
                            jnp.where(hvs[1] == rbest, his[1], bigi16)),
                jnp.minimum(jnp.where(hvs[2] == rbest, his[2], bigi16),
                            jnp.where(hvs[3] == rbest, his[3], bigi16)))
            ibest = jnp.min(mm)
            lwin = ibest & (LANES - 1)
            selw = lane == lwin
            for ch in range(NCH):
                upd = selw & (hvs[ch] == rbest) & (his[ch] == ibest)
                ptrs[ch] = ptrs[ch] + upd.astype(jnp.int32)

            tt = t & (LANES - 1)
            tb = t >> 4
            sel = lane == tt
            ov0 = jnp.where(sel & (tb == 0), rbest, ov0)
            ov1 = jnp.where(sel & (tb == 1), rbest, ov1)
            ov2 = jnp.where(sel & (tb == 2), rbest, ov2)
            ov3 = jnp.where(sel & (tb == 3), rbest, ov3)
            oi0 = jnp.where(sel & (tb == 0), ibest, oi0)
            oi1 = jnp.where(sel & (tb == 1), ibest, oi1)
            oi2 = jnp.where(sel & (tb == 2), ibest, oi2)
            oi3 = jnp.where(sel & (tb == 3), ibest, oi3)
            return (ov0, ov1, ov2, ov3, oi0, oi1, oi2, oi3,
                    ptrs[0], ptrs[1], ptrs[2], ptrs[3])

        st0 = (neginf16, neginf16, neginf16, neginf16,
               zero16i, zero16i, zero16i, zero16i,
               zero16i, zero16i, zero16i, zero16i)
        out = lax.fori_loop(0, KTOP, ext_body, st0)
        ov0, ov1, ov2, ov3, oi0, oi1, oi2, oi3 = out[:8]

        outv[pl.ds(0, LANES)] = ov0
        outv[pl.ds(LANES, LANES)] = ov1
        outv[pl.ds(2 * LANES, LANES)] = ov2
        outv[pl.ds(3 * LANES, LANES)] = ov3
        outi[pl.ds(0, LANES)] = oi0
        outi[pl.ds(LANES, LANES)] = oi1
        outi[pl.ds(2 * LANES, LANES)] = oi2
        outi[pl.ds(3 * LANES, LANES)] = oi3
        pltpu.sync_copy(outv, v_hbm.at[row])
        pltpu.sync_copy(outi, i_hbm.at[row])
        return 0

    lax.fori_loop(0, RPW, do_row, 0)


def kernel(x):
    mesh = plsc.VectorSubcoreMesh(
        core_axis_name="c", subcore_axis_name="s", num_cores=NC, num_subcores=NS)
    f = pl.kernel(
        _sc_body,
        out_type=(
            jax.ShapeDtypeStruct((ROWS, KTOP), jnp.float32),
            jax.ShapeDtypeStruct((ROWS, KTOP), jnp.int32),
        ),
        mesh=mesh,
        compiler_params=pltpu.CompilerParams(needs_layout_passes=False, use_tc_tiling_on_sc=True),
        scratch_types=[
            pltpu.VMEM((2 * N,), jnp.float32),
            pltpu.VMEM((NCH * CL, LANES), jnp.float32),
            pltpu.VMEM((NCH * CL, LANES), jnp.int32),
            pltpu.VMEM((KTOP,), jnp.float32),
            pltpu.VMEM((KTOP,), jnp.int32),
            pltpu.SemaphoreType.DMA,
        ],
    )
    return f(x)


# speedup vs baseline: 16.3505x; 1.1090x over previous
import jax
import jax.numpy as jnp
from jax import lax
from jax.experimental import pallas as pl
from jax.experimental.pallas import tpu as pltpu
from jax.experimental.pallas import tpu_sc as plsc

KTOP = 64
N = 32768
ROWS = 128
LANES = 16
NV = N // LANES
GB = 64
NG = NV // GB
NCH = 4
CL = 32
NC, NS = 2, 16
NW = NC * NS
RPW = ROWS // NW
UNROLL = 16

_NEG_INF = float("-inf")
_BIG_I = 2**30


def _sc_body(x_hbm, v_hbm, i_hbm, xb, vbuf, ibuf, cref, outv, outi, sem):
    wid = lax.axis_index("s") * NC + lax.axis_index("c")
    lane = lax.iota(jnp.int32, LANES)
    zero16i = jnp.zeros((LANES,), jnp.int32)
    neginf16 = jnp.full((LANES,), _NEG_INF, jnp.float32)
    bigi16 = jnp.full((LANES,), _BIG_I, jnp.int32)

    row0 = wid * RPW
    pltpu.async_copy(x_hbm.at[row0], xb.at[pl.ds(0, N)], sem)

    def do_row(r, _):
        row = row0 + r
        base = (r & 1) * N
        pltpu.make_async_copy(
            x_hbm.at[row], xb.at[pl.ds(base, N)], sem).wait()

        @pl.when(r + 1 < RPW)
        def _():
            nbase = ((r + 1) & 1) * N
            pltpu.async_copy(
                x_hbm.at[row + 1], xb.at[pl.ds(nbase, N)], sem)

        def init_body(j, _):
            vbuf[j] = neginf16
            return 0
        lax.fori_loop(0, NCH * CL, init_body, 0)

        def collect(tvec):
            def col_body(i0, curs):
                curs = list(curs)
                bi = i0 * UNROLL
                vs = [xb[pl.ds(base + (bi + u) * LANES, LANES)]
                      for u in range(UNROLL)]
                for u in range(UNROLL):
                    ch = u & (NCH - 1)
                    v = vs[u]
                    msk = v >= tvec
                    idx = lane + (bi + u) * LANES
                    addr = (curs[ch] & (CL - 1)) + ch * CL
                    plsc.store_scatter(ibuf, [addr, lane], idx, mask=msk)
                    curs[ch] = curs[ch] + msk.astype(jnp.int32)
                return tuple(curs)
            return lax.fori_loop(0, NV // UNROLL, col_body, (zero16i,) * NCH)

        tspec = jnp.full((LANES,), 2.7, jnp.float32)
        curs = collect(tspec)
        csum = curs[0] + curs[1] + curs[2] + curs[3]
        cmax = jnp.maximum(jnp.maximum(curs[0], curs[1]),
                           jnp.maximum(curs[2], curs[3]))
        ok = (jnp.sum(csum) >= KTOP) & (jnp.max(cmax) <= CL)
        for ch in range(NCH):
            cref[ch] = curs[ch]

        @pl.when(jnp.logical_not(ok))
        def _():
            def g_body(g, tops):
                t0, t1, t2, t3 = tops
                m = xb[pl.ds(base + g * (GB * LANES), LANES)]
                for j in range(1, GB):
                    m = jnp.maximum(
                        m, xb[pl.ds(base + g * (GB * LANES) + j * LANES, LANES)])
                hi = jnp.maximum(t0, m)
                m = jnp.minimum(t0, m)
                t0 = hi
                hi = jnp.maximum(t1, m)
                m = jnp.minimum(t1, m)
                t1 = hi
                hi = jnp.maximum(t2, m)
                m = jnp.minimum(t2, m)
                t2 = hi
                t3 = jnp.maximum(t3, m)
                return t0, t1, t2, t3

            _, _, _, t3 = lax.fori_loop(
                0, NG, g_body, (neginf16, neginf16, neginf16, neginf16))
            tvec = jnp.zeros((LANES,), jnp.float32) + jnp.min(t3)
            curs2 = collect(tvec)
            for ch in range(NCH):
                cref[ch] = curs2[ch]

        curs = tuple(cref[ch] for ch in range(NCH))

        for ch in range(NCH):
            cur_c = curs[ch]
            lmax_c = jnp.minimum(jnp.max(cur_c), CL)

            def mat_body(j, _, ch=ch, cur_c=cur_c):
                idx = ibuf[ch * CL + j]
                ok = cur_c > j
                v = plsc.load_gather(xb, [base + idx], mask=ok)
                vbuf[ch * CL + j] = jnp.where(ok, v, _NEG_INF)
                return 0

            lax.fori_loop(0, lmax_c, mat_body, 0)

            def sweep(_, carry, ch=ch, lmax_c=lmax_c):
                def ce(j, c):
                    a = ch * CL + j
                    va = vbuf[a]
                    vb = vbuf[a + 1]
                    ia = ibuf[a]
                    ib = ibuf[a + 1]
                    sw = vb > va
                    vbuf[a] = jnp.where(sw, vb, va)
                    vbuf[a + 1] = jnp.where(sw, va, vb)
                    ibuf[a] = jnp.where(sw, ib, ia)
                    ibuf[a + 1] = jnp.where(sw, ia, ib)
                    return c
                return lax.fori_loop(0, lmax_c - 1, ce, carry)

            lax.fori_loop(0, lmax_c, sweep, 0)

        def ext_body(t, st):
            (ov0, ov1, ov2, ov3, oi0, oi1, oi2, oi3,
             p0, p1, p2, p3) = st
            ptrs = [p0, p1, p2, p3]
            hvs, his = [], []
            for ch in range(NCH):
                inb = ptrs[ch] < CL
                hv = plsc.load_gather(vbuf, [ptrs[ch] + ch * CL, lane],
                                      mask=inb)
                hi = plsc.load_gather(ibuf, [ptrs[ch] + ch * CL, lane],
                                      mask=inb)
                hvs.append(jnp.where(inb, hv, _NEG_INF))
                his.append(hi)
            hvm = jnp.maximum(jnp.maximum(hvs[0], hvs[1]),
                              jnp.maximum(hvs[2], hvs[3]))
            rbest = jnp.max(hvm)
            mm = jnp.minimum(
                jnp.minimum(jnp.where(hvs[0] == rbest, his[0], bigi16),
                            jnp.where(hvs[1] == rbest, his[1], bigi16)),
                jnp.minimum(jnp.where(hvs[2] == rbest, his[2], bigi16),
                            jnp.where(hvs[3] == rbest, his[3], bigi16)))
            ibest = jnp.min(mm)
            lwin = ibest & (LANES - 1)
            selw = lane == lwin
            for ch in range(NCH):
                upd = selw & (hvs[ch] == rbest) & (his[ch] == ibest)
                ptrs[ch] = ptrs[ch] + upd.astype(jnp.int32)

            tt = t & (LANES - 1)
            tb = t >> 4
            sel = lane == tt
            ov0 = jnp.where(sel & (tb == 0), rbest, ov0)
            ov1 = jnp.where(sel & (tb == 1), rbest, ov1)
            ov2 = jnp.where(sel & (tb == 2), rbest, ov2)
            ov3 = jnp.where(sel & (tb == 3), rbest, ov3)
            oi0 = jnp.where(sel & (tb == 0), ibest, oi0)
            oi1 = jnp.where(sel & (tb == 1), ibest, oi1)
            oi2 = jnp.where(sel & (tb == 2), ibest, oi2)
            oi3 = jnp.where(sel & (tb == 3), ibest, oi3)
            return (ov0, ov1, ov2, ov3, oi0, oi1, oi2, oi3,
                    ptrs[0], ptrs[1], ptrs[2], ptrs[3])

        st0 = (neginf16, neginf16, neginf16, neginf16,
               zero16i, zero16i, zero16i, zero16i,
               zero16i, zero16i, zero16i, zero16i)
        out = lax.fori_loop(0, KTOP, ext_body, st0)
        ov0, ov1, ov2, ov3, oi0, oi1, oi2, oi3 = out[:8]

        outv[pl.ds(0, LANES)] = ov0
        outv[pl.ds(LANES, LANES)] = ov1
        outv[pl.ds(2 * LANES, LANES)] = ov2
        outv[pl.ds(3 * LANES, LANES)] = ov3
        outi[pl.ds(0, LANES)] = oi0
        outi[pl.ds(LANES, LANES)] = oi1
        outi[pl.ds(2 * LANES, LANES)] = oi2
        outi[pl.ds(3 * LANES, LANES)] = oi3
        pltpu.sync_copy(outv, v_hbm.at[row])
        pltpu.sync_copy(outi, i_hbm.at[row])
        return 0

    lax.fori_loop(0, RPW, do_row, 0)


def kernel(x):
    mesh = plsc.VectorSubcoreMesh(
        core_axis_name="c", subcore_axis_name="s", num_cores=NC, num_subcores=NS)
    f = pl.kernel(
        _sc_body,
        out_type=(
            jax.ShapeDtypeStruct((ROWS, KTOP), jnp.float32),
            jax.ShapeDtypeStruct((ROWS, KTOP), jnp.int32),
        ),
        mesh=mesh,
        compiler_params=pltpu.CompilerParams(needs_layout_passes=False, use_tc_tiling_on_sc=True),
        scratch_types=[
            pltpu.VMEM((2 * N,), jnp.float32),
            pltpu.VMEM((NCH * CL, LANES), jnp.float32),
            pltpu.VMEM((NCH * CL, LANES), jnp.int32),
            pltpu.VMEM((NCH, LANES), jnp.int32),
            pltpu.VMEM((KTOP,), jnp.float32),
            pltpu.VMEM((KTOP,), jnp.int32),
            pltpu.SemaphoreType.DMA,
        ],
    )
    return f(x)


# speedup vs baseline: 16.3732x; 1.0014x over previous
import jax
import jax.numpy as jnp
from jax import lax
from jax.experimental import pallas as pl
from jax.experimental.pallas import tpu as pltpu
from jax.experimental.pallas import tpu_sc as plsc

KTOP = 64
N = 32768
ROWS = 128
LANES = 16
NV = N // LANES
GB = 64
NG = NV // GB
NCH = 4
CL = 32
NC, NS = 2, 16
NW = NC * NS
RPW = ROWS // NW
UNROLL = 16

_NEG_INF = float("-inf")
_BIG_I = 2**30


def _sc_body(x_hbm, v_hbm, i_hbm, xb, vbuf, ibuf, cref, outv, outi, sem):
    wid = lax.axis_index("s") * NC + lax.axis_index("c")
    lane = lax.iota(jnp.int32, LANES)
    zero16i = jnp.zeros((LANES,), jnp.int32)
    neginf16 = jnp.full((LANES,), _NEG_INF, jnp.float32)
    bigi16 = jnp.full((LANES,), _BIG_I, jnp.int32)

    row0 = wid * RPW
    pltpu.async_copy(x_hbm.at[row0], xb.at[pl.ds(0, N)], sem)

    def do_row(r, _):
        row = row0 + r
        base = (r & 1) * N
        pltpu.make_async_copy(
            x_hbm.at[row], xb.at[pl.ds(base, N)], sem).wait()

        @pl.when(r + 1 < RPW)
        def _():
            nbase = ((r + 1) & 1) * N
            pltpu.async_copy(
                x_hbm.at[row + 1], xb.at[pl.ds(nbase, N)], sem)

        def init_body(j, _):
            vbuf[j] = neginf16
            return 0
        lax.fori_loop(0, NCH * CL, init_body, 0)

        def collect(tvec):
            def col_body(i0, curs):
                curs = list(curs)
                bi = i0 * UNROLL
                vs = [xb[pl.ds(base + (bi + u) * LANES, LANES)]
                      for u in range(UNROLL)]
                for u in range(UNROLL):
                    ch = u & (NCH - 1)
                    v = vs[u]
                    msk = v >= tvec
                    idx = lane + (bi + u) * LANES
                    addr = (curs[ch] & (CL - 1)) + ch * CL
                    plsc.store_scatter(ibuf, [addr, lane], idx, mask=msk)
                    curs[ch] = curs[ch] + msk.astype(jnp.int32)
                return tuple(curs)
            return lax.fori_loop(0, NV // UNROLL, col_body, (zero16i,) * NCH)

        tspec = jnp.full((LANES,), 2.7, jnp.float32)
        curs = collect(tspec)
        csum = curs[0] + curs[1] + curs[2] + curs[3]
        cmax = jnp.maximum(jnp.maximum(curs[0], curs[1]),
                           jnp.maximum(curs[2], curs[3]))
        ok = (jnp.sum(csum) >= KTOP) & (jnp.max(cmax) <= CL)
        for ch in range(NCH):
            cref[ch] = curs[ch]

        @pl.when(jnp.logical_not(ok))
        def _():
            def g_body(g, tops):
                t0, t1, t2, t3 = tops
                m = xb[pl.ds(base + g * (GB * LANES), LANES)]
                for j in range(1, GB):
                    m = jnp.maximum(
                        m, xb[pl.ds(base + g * (GB * LANES) + j * LANES, LANES)])
                hi = jnp.maximum(t0, m)
                m = jnp.minimum(t0, m)
                t0 = hi
                hi = jnp.maximum(t1, m)
                m = jnp.minimum(t1, m)
                t1 = hi
                hi = jnp.maximum(t2, m)
                m = jnp.minimum(t2, m)
                t2 = hi
                t3 = jnp.maximum(t3, m)
                return t0, t1, t2, t3

            _, _, _, t3 = lax.fori_loop(
                0, NG, g_body, (neginf16, neginf16, neginf16, neginf16))
            tvec = jnp.zeros((LANES,), jnp.float32) + jnp.min(t3)
            curs2 = collect(tvec)
            for ch in range(NCH):
                cref[ch] = curs2[ch]

        curs = tuple(cref[ch] for ch in range(NCH))

        for ch in range(NCH):
            cur_c = curs[ch]
            lmax_c = jnp.minimum(jnp.max(cur_c), CL)

            def mat_body(j, _, ch=ch, cur_c=cur_c):
                idx = ibuf[ch * CL + j]
                ok = cur_c > j
                v = plsc.load_gather(xb, [base + idx], mask=ok)
                vbuf[ch * CL + j] = jnp.where(ok, v, _NEG_INF)
                return 0

            lax.fori_loop(0, lmax_c, mat_body, 0)

            def sweep(_, carry, ch=ch, lmax_c=lmax_c):
                def ce(j, c):
                    a = ch * CL + j
                    va = vbuf[a]
                    vb = vbuf[a + 1]
                    ia = ibuf[a]
                    ib = ibuf[a + 1]
                    sw = vb > va
                    vbuf[a] = jnp.where(sw, vb, va)
                    vbuf[a + 1] = jnp.where(sw, va, vb)
                    ibuf[a] = jnp.where(sw, ib, ia)
                    ibuf[a + 1] = jnp.where(sw, ia, ib)
                    return c
                return lax.fori_loop(0, lmax_c - 1, ce, carry)

            lax.fori_loop(0, lmax_c, sweep, 0)

        def ext_body(t, st):
            (ov0, ov1, ov2, ov3, oi0, oi1, oi2, oi3,
             p0, p1, p2, p3) = st
            ptrs = [p0, p1, p2, p3]
            hvs, his = [], []
            for ch in range(NCH):
                inb = ptrs[ch] < CL
                hv = plsc.load_gather(vbuf, [ptrs[ch] + ch * CL, lane],
                                      mask=inb)
                hi = plsc.load_gather(ibuf, [ptrs[ch] + ch * CL, lane],
                                      mask=inb)
                hvs.append(jnp.where(inb, hv, _NEG_INF))
                his.append(hi)
            hvm = jnp.maximum(jnp.maximum(hvs[0], hvs[1]),
                              jnp.maximum(hvs[2], hvs[3]))
            rbest = jnp.max(hvm)
            mm = jnp.minimum(
                jnp.minimum(jnp.where(hvs[0] == rbest, his[0], bigi16),
                            jnp.where(hvs[1] == rbest, his[1], bigi16)),
                jnp.minimum(jnp.where(hvs[2] == rbest, his[2], bigi16),
                            jnp.where(hvs[3] == rbest, his[3], bigi16)))
            ibest = jnp.min(mm)
            lwin = ibest & (LANES - 1)
            selw = lane == lwin
            for ch in range(NCH):
                upd = selw & (hvs[ch] == rbest) & (his[ch] == ibest)
                ptrs[ch] = ptrs[ch] + upd.astype(jnp.int32)

            tt = t & (LANES - 1)
            tb = t >> 4
            sel = lane == tt
            ov0 = jnp.where(sel & (tb == 0), rbest, ov0)
            ov1 = jnp.where(sel & (tb == 1), rbest, ov1)
            ov2 = jnp.where(sel & (tb == 2), rbest, ov2)
            ov3 = jnp.where(sel & (tb == 3), rbest, ov3)
            oi0 = jnp.where(sel & (tb == 0), ibest, oi0)
            oi1 = jnp.where(sel & (tb == 1), ibest, oi1)
            oi2 = jnp.where(sel & (tb == 2), ibest, oi2)
            oi3 = jnp.where(sel & (tb == 3), ibest, oi3)
            return (ov0, ov1, ov2, ov3, oi0, oi1, oi2, oi3,
                    ptrs[0], ptrs[1], ptrs[2], ptrs[3])

        st0 = (neginf16, neginf16, neginf16, neginf16,
               zero16i, zero16i, zero16i, zero16i,
               zero16i, zero16i, zero16i, zero16i)
        out = lax.fori_loop(0, KTOP, ext_body, st0)
        ov0, ov1, ov2, ov3, oi0, oi1, oi2, oi3 = out[:8]

        outv[pl.ds(0, LANES)] = ov0
        outv[pl.ds(LANES, LANES)] = ov1
        outv[pl.ds(2 * LANES, LANES)] = ov2
        outv[pl.ds(3 * LANES, LANES)] = ov3
        outi[pl.ds(0, LANES)] = oi0
        outi[pl.ds(LANES, LANES)] = oi1
        outi[pl.ds(2 * LANES, LANES)] = oi2
        outi[pl.ds(3 * LANES, LANES)] = oi3
        pltpu.sync_copy(outv, v_hbm.at[row])
        pltpu.sync_copy(outi, i_hbm.at[row])
        return 0

    lax.fori_loop(0, RPW, do_row, 0)


def kernel(x):
    mesh = plsc.VectorSubcoreMesh(
        core_axis_name="c", subcore_axis_name="s", num_cores=NC, num_subcores=NS)
    f = pl.kernel(
        _sc_body,
        out_type=(
            jax.ShapeDtypeStruct((ROWS, KTOP), jnp.float32),
            jax.ShapeDtypeStruct((ROWS, KTOP), jnp.int32),
        ),
        mesh=mesh,
        compiler_params=pltpu.CompilerParams(needs_layout_passes=False, use_tc_tiling_on_sc=True, skip_device_barrier=True),
        scratch_types=[
            pltpu.VMEM((2 * N,), jnp.float32),
            pltpu.VMEM((NCH * CL, LANES), jnp.float32),
            pltpu.VMEM((NCH * CL, LANES), jnp.int32),
            pltpu.VMEM((NCH, LANES), jnp.int32),
            pltpu.VMEM((KTOP,), jnp.float32),
            pltpu.VMEM((KTOP,), jnp.int32),
            pltpu.SemaphoreType.DMA,
        ],
    )
    return f(x)


# speedup vs baseline: 16.5443x; 1.0105x over previous
import jax
import jax.numpy as jnp
from jax import lax
from jax.experimental import pallas as pl
from jax.experimental.pallas import tpu as pltpu
from jax.experimental.pallas import tpu_sc as plsc

KTOP = 64
N = 32768
ROWS = 128
LANES = 16
NV = N // LANES
GB = 64
NG = NV // GB
NCH = 4
CL = 32
NC, NS = 2, 16
NW = NC * NS
RPW = ROWS // NW
UNROLL = 16

_NEG_INF = float("-inf")
_BIG_I = 2**30


def _sc_body(x_hbm, v_hbm, i_hbm, xb, vbuf, ibuf, cref, outv, outi, sem):
    wid = lax.axis_index("s") * NC + lax.axis_index("c")
    lane = lax.iota(jnp.int32, LANES)
    zero16i = jnp.zeros((LANES,), jnp.int32)
    neginf16 = jnp.full((LANES,), _NEG_INF, jnp.float32)
    zero16f = jnp.zeros((LANES,), jnp.float32)
    bigi16 = jnp.full((LANES,), _BIG_I, jnp.int32)

    row0 = wid * RPW
    pltpu.async_copy(x_hbm.at[row0], xb.at[pl.ds(0, N)], sem)

    def do_row(r, _):
        row = row0 + r
        base = (r & 1) * N
        def init_body(j, _):
            vbuf[j] = neginf16
            return 0
        lax.fori_loop(0, NCH * CL, init_body, 0)

        pltpu.make_async_copy(
            x_hbm.at[row], xb.at[pl.ds(base, N)], sem).wait()

        @pl.when(r + 1 < RPW)
        def _():
            nbase = ((r + 1) & 1) * N
            pltpu.async_copy(
                x_hbm.at[row + 1], xb.at[pl.ds(nbase, N)], sem)

        def collect(tvec):
            def col_body(i0, curs):
                curs = list(curs)
                bi = i0 * UNROLL
                vs = [xb[pl.ds(base + (bi + u) * LANES, LANES)]
                      for u in range(UNROLL)]
                for u in range(UNROLL):
                    ch = u & (NCH - 1)
                    v = vs[u]
                    msk = v >= tvec
                    idx = lane + (bi + u) * LANES
                    addr = (curs[ch] & (CL - 1)) + ch * CL
                    plsc.store_scatter(ibuf, [addr, lane], idx, mask=msk)
                    curs[ch] = curs[ch] + msk.astype(jnp.int32)
                return tuple(curs)
            return lax.fori_loop(0, NV // UNROLL, col_body, (zero16i,) * NCH)

        tspec = jnp.full((LANES,), 2.7, jnp.float32)
        curs = collect(tspec)
        csum = curs[0] + curs[1] + curs[2] + curs[3]
        cmax = jnp.maximum(jnp.maximum(curs[0], curs[1]),
                           jnp.maximum(curs[2], curs[3]))
        ok = (jnp.sum(csum) >= KTOP) & (jnp.max(cmax) <= CL)
        for ch in range(NCH):
            cref[ch] = curs[ch]

        @pl.when(jnp.logical_not(ok))
        def _():
            def g_body(g, tops):
                t0, t1, t2, t3 = tops
                m = xb[pl.ds(base + g * (GB * LANES), LANES)]
                for j in range(1, GB):
                    m = jnp.maximum(
                        m, xb[pl.ds(base + g * (GB * LANES) + j * LANES, LANES)])
                hi = jnp.maximum(t0, m)
                m = jnp.minimum(t0, m)
                t0 = hi
                hi = jnp.maximum(t1, m)
                m = jnp.minimum(t1, m)
                t1 = hi
                hi = jnp.maximum(t2, m)
                m = jnp.minimum(t2, m)
                t2 = hi
                t3 = jnp.maximum(t3, m)
                return t0, t1, t2, t3

            _, _, _, t3 = lax.fori_loop(
                0, NG, g_body, (neginf16, neginf16, neginf16, neginf16))
            tvec = jnp.zeros((LANES,), jnp.float32) + jnp.min(t3)
            curs2 = collect(tvec)
            for ch in range(NCH):
                cref[ch] = curs2[ch]

        curs = tuple(cref[ch] for ch in range(NCH))

        for ch in range(NCH):
            cur_c = curs[ch]
            lmax_c = jnp.minimum(jnp.max(cur_c), CL)

            def mat_body(j, _, ch=ch, cur_c=cur_c):
                idx = ibuf[ch * CL + j]
                ok = cur_c > j
                v = plsc.load_gather(xb, [base + idx], mask=ok)
                vbuf[ch * CL + j] = jnp.where(ok, v, _NEG_INF)
                return 0

            lax.fori_loop(0, lmax_c, mat_body, 0)

            def sweep(_, carry, ch=ch, lmax_c=lmax_c):
                def ce(j, c):
                    a = ch * CL + j
                    va = vbuf[a]
                    vb = vbuf[a + 1]
                    ia = ibuf[a]
                    ib = ibuf[a + 1]
                    sw = vb > va
                    vbuf[a] = jnp.where(sw, vb, va)
                    vbuf[a + 1] = jnp.where(sw, va, vb)
                    ibuf[a] = jnp.where(sw, ib, ia)
                    ibuf[a + 1] = jnp.where(sw, ia, ib)
                    return c
                return lax.fori_loop(0, lmax_c - 1, ce, carry)

            lax.fori_loop(0, lmax_c, sweep, 0)

        lane0 = lane == 0
        def ext_body(t, st):
            p0, p1, p2, p3 = st
            ptrs = [p0, p1, p2, p3]
            hvs, his = [], []
            for ch in range(NCH):
                inb = ptrs[ch] < CL
                hv = plsc.load_gather(vbuf, [ptrs[ch] + ch * CL, lane],
                                      mask=inb)
                hi = plsc.load_gather(ibuf, [ptrs[ch] + ch * CL, lane],
                                      mask=inb)
                hvs.append(jnp.where(inb, hv, _NEG_INF))
                his.append(hi)
            hvm = jnp.maximum(jnp.maximum(hvs[0], hvs[1]),
                              jnp.maximum(hvs[2], hvs[3]))
            rbest = jnp.max(hvm)
            mm = jnp.minimum(
                jnp.minimum(jnp.where(hvs[0] == rbest, his[0], bigi16),
                            jnp.where(hvs[1] == rbest, his[1], bigi16)),
                jnp.minimum(jnp.where(hvs[2] == rbest, his[2], bigi16),
                            jnp.where(hvs[3] == rbest, his[3], bigi16)))
            ibest = jnp.min(mm)
            lwin = ibest & (LANES - 1)
            selw = lane == lwin
            for ch in range(NCH):
                upd = selw & (hvs[ch] == rbest) & (his[ch] == ibest)
                ptrs[ch] = ptrs[ch] + upd.astype(jnp.int32)

            tvecidx = zero16i + t
            plsc.store_scatter(outv, [tvecidx], zero16f + rbest, mask=lane0)
            plsc.store_scatter(outi, [tvecidx], zero16i + ibest, mask=lane0)
            return (ptrs[0], ptrs[1], ptrs[2], ptrs[3])

        st0 = (zero16i, zero16i, zero16i, zero16i)
        lax.fori_loop(0, KTOP, ext_body, st0)

        pltpu.sync_copy(outv, v_hbm.at[row])
        pltpu.sync_copy(outi, i_hbm.at[row])
        return 0

    lax.fori_loop(0, RPW, do_row, 0)


def kernel(x):
    mesh = plsc.VectorSubcoreMesh(
        core_axis_name="c", subcore_axis_name="s", num_cores=NC, num_subcores=NS)
    f = pl.kernel(
        _sc_body,
        out_type=(
            jax.ShapeDtypeStruct((ROWS, KTOP), jnp.float32),
            jax.ShapeDtypeStruct((ROWS, KTOP), jnp.int32),
        ),
        mesh=mesh,
        compiler_params=pltpu.CompilerParams(needs_layout_passes=False, use_tc_tiling_on_sc=True),
        scratch_types=[
            pltpu.VMEM((2 * N,), jnp.float32),
            pltpu.VMEM((NCH * CL, LANES), jnp.float32),
            pltpu.VMEM((NCH * CL, LANES), jnp.int32),
            pltpu.VMEM((NCH, LANES), jnp.int32),
            pltpu.VMEM((KTOP,), jnp.float32),
            pltpu.VMEM((KTOP,), jnp.int32),
            pltpu.SemaphoreType.DMA,
        ],
    )
    return f(x)


# speedup vs baseline: 16.7643x; 1.0133x over previous
import jax
import jax.numpy as jnp
from jax import lax
from jax.experimental import pallas as pl
from jax.experimental.pallas import tpu as pltpu
from jax.experimental.pallas import tpu_sc as plsc

KTOP = 64
N = 32768
ROWS = 128
LANES = 16
NV = N // LANES
GB = 64
NG = NV // GB
NCH = 2
CL = 32
NC, NS = 2, 16
NW = NC * NS
RPW = ROWS // NW
UNROLL = 16

_NEG_INF = float("-inf")
_BIG_I = 2**30


def _sc_body(x_hbm, v_hbm, i_hbm, xb, vbuf, ibuf, cref, outv, outi, sem):
    wid = lax.axis_index("s") * NC + lax.axis_index("c")
    lane = lax.iota(jnp.int32, LANES)
    zero16i = jnp.zeros((LANES,), jnp.int32)
    neginf16 = jnp.full((LANES,), _NEG_INF, jnp.float32)
    zero16f = jnp.zeros((LANES,), jnp.float32)
    bigi16 = jnp.full((LANES,), _BIG_I, jnp.int32)

    row0 = wid * RPW
    pltpu.async_copy(x_hbm.at[row0], xb.at[pl.ds(0, N)], sem)

    def do_row(r, _):
        row = row0 + r
        base = (r & 1) * N
        def init_body(j, _):
            vbuf[j] = neginf16
            return 0
        lax.fori_loop(0, NCH * CL, init_body, 0)

        pltpu.make_async_copy(
            x_hbm.at[row], xb.at[pl.ds(base, N)], sem).wait()

        @pl.when(r + 1 < RPW)
        def _():
            nbase = ((r + 1) & 1) * N
            pltpu.async_copy(
                x_hbm.at[row + 1], xb.at[pl.ds(nbase, N)], sem)

        def collect(tvec):
            def col_body(i0, curs):
                curs = list(curs)
                bi = i0 * UNROLL
                vs = [xb[pl.ds(base + (bi + u) * LANES, LANES)]
                      for u in range(UNROLL)]
                for u in range(UNROLL):
                    ch = u & (NCH - 1)
                    v = vs[u]
                    msk = v >= tvec
                    idx = lane + (bi + u) * LANES
                    addr = (curs[ch] & (CL - 1)) + ch * CL
                    plsc.store_scatter(ibuf, [addr, lane], idx, mask=msk)
                    curs[ch] = curs[ch] + msk.astype(jnp.int32)
                return tuple(curs)
            return lax.fori_loop(0, NV // UNROLL, col_body, (zero16i,) * NCH)

        tspec = jnp.full((LANES,), 2.7, jnp.float32)
        curs = collect(tspec)
        csum = curs[0]
        cmax = curs[0]
        for ch in range(1, NCH):
            csum = csum + curs[ch]
            cmax = jnp.maximum(cmax, curs[ch])
        ok = (jnp.sum(csum) >= KTOP) & (jnp.max(cmax) <= CL)
        for ch in range(NCH):
            cref[ch] = curs[ch]

        @pl.when(jnp.logical_not(ok))
        def _():
            def g_body(g, tops):
                t0, t1, t2, t3 = tops
                m = xb[pl.ds(base + g * (GB * LANES), LANES)]
                for j in range(1, GB):
                    m = jnp.maximum(
                        m, xb[pl.ds(base + g * (GB * LANES) + j * LANES, LANES)])
                hi = jnp.maximum(t0, m)
                m = jnp.minimum(t0, m)
                t0 = hi
                hi = jnp.maximum(t1, m)
                m = jnp.minimum(t1, m)
                t1 = hi
                hi = jnp.maximum(t2, m)
                m = jnp.minimum(t2, m)
                t2 = hi
                t3 = jnp.maximum(t3, m)
                return t0, t1, t2, t3

            _, _, _, t3 = lax.fori_loop(
                0, NG, g_body, (neginf16, neginf16, neginf16, neginf16))
            tvec = jnp.zeros((LANES,), jnp.float32) + jnp.min(t3)
            curs2 = collect(tvec)
            for ch in range(NCH):
                cref[ch] = curs2[ch]

        curs = tuple(cref[ch] for ch in range(NCH))

        for ch in range(NCH):
            cur_c = curs[ch]
            lmax_c = jnp.minimum(jnp.max(cur_c), CL)

            def mat_body(j, _, ch=ch, cur_c=cur_c):
                idx = ibuf[ch * CL + j]
                ok = cur_c > j
                v = plsc.load_gather(xb, [base + idx], mask=ok)
                vbuf[ch * CL + j] = jnp.where(ok, v, _NEG_INF)
                return 0

            lax.fori_loop(0, lmax_c, mat_body, 0)

            def sweep(_, carry, ch=ch, lmax_c=lmax_c):
                def ce(j, c):
                    a = ch * CL + j
                    va = vbuf[a]
                    vb = vbuf[a + 1]
                    ia = ibuf[a]
                    ib = ibuf[a + 1]
                    sw = vb > va
                    vbuf[a] = jnp.where(sw, vb, va)
                    vbuf[a + 1] = jnp.where(sw, va, vb)
                    ibuf[a] = jnp.where(sw, ib, ia)
                    ibuf[a + 1] = jnp.where(sw, ia, ib)
                    return c
                return lax.fori_loop(0, lmax_c - 1, ce, carry)

            lax.fori_loop(0, lmax_c, sweep, 0)

        lane0 = lane == 0
        def ext_body(t, st):
            ptrs = list(st)
            hvs, his = [], []
            for ch in range(NCH):
                inb = ptrs[ch] < CL
                hv = plsc.load_gather(vbuf, [ptrs[ch] + ch * CL, lane],
                                      mask=inb)
                hi = plsc.load_gather(ibuf, [ptrs[ch] + ch * CL, lane],
                                      mask=inb)
                hvs.append(jnp.where(inb, hv, _NEG_INF))
                his.append(hi)
            hvm = hvs[0]
            for ch in range(1, NCH):
                hvm = jnp.maximum(hvm, hvs[ch])
            rbest = jnp.max(hvm)
            mm = jnp.where(hvs[0] == rbest, his[0], bigi16)
            for ch in range(1, NCH):
                mm = jnp.minimum(mm, jnp.where(hvs[ch] == rbest, his[ch], bigi16))
            ibest = jnp.min(mm)
            lwin = ibest & (LANES - 1)
            selw = lane == lwin
            for ch in range(NCH):
                upd = selw & (hvs[ch] == rbest) & (his[ch] == ibest)
                ptrs[ch] = ptrs[ch] + upd.astype(jnp.int32)

            tvecidx = zero16i + t
            plsc.store_scatter(outv, [tvecidx], zero16f + rbest, mask=lane0)
            plsc.store_scatter(outi, [tvecidx], zero16i + ibest, mask=lane0)
            return tuple(ptrs)

        st0 = (zero16i,) * NCH
        lax.fori_loop(0, KTOP, ext_body, st0)

        pltpu.sync_copy(outv, v_hbm.at[row])
        pltpu.sync_copy(outi, i_hbm.at[row])
        return 0

    lax.fori_loop(0, RPW, do_row, 0)


def kernel(x):
    mesh = plsc.VectorSubcoreMesh(
        core_axis_name="c", subcore_axis_name="s", num_cores=NC, num_subcores=NS)
    f = pl.kernel(
        _sc_body,
        out_type=(
            jax.ShapeDtypeStruct((ROWS, KTOP), jnp.float32),
            jax.ShapeDtypeStruct((ROWS, KTOP), jnp.int32),
        ),
        mesh=mesh,
        compiler_params=pltpu.CompilerParams(needs_layout_passes=False, use_tc_tiling_on_sc=True),
        scratch_types=[
            pltpu.VMEM((2 * N,), jnp.float32),
            pltpu.VMEM((NCH * CL, LANES), jnp.float32),
            pltpu.VMEM((NCH * CL, LANES), jnp.int32),
            pltpu.VMEM((NCH, LANES), jnp.int32),
            pltpu.VMEM((KTOP,), jnp.float32),
            pltpu.VMEM((KTOP,), jnp.int32),
            pltpu.SemaphoreType.DMA,
        ],
    )
    return f(x)
